# baseline (speedup 1.0000x reference)
import jax
import jax.numpy as jnp
from jax.experimental import pallas as pl
from jax.experimental.pallas import tpu as pltpu

_BN_EPS = 1e-5
_BR = 400


def _support_body(x_ref, w_ref, o_ref):
    o_ref[...] = jnp.dot(x_ref[...], w_ref[...],
                         preferred_element_type=jnp.float32)


def _spmm_body(adj_ref, s_ref, o_ref):
    o_ref[...] = jnp.dot(adj_ref[...], s_ref[...],
                         preferred_element_type=jnp.float32)


def _bn_tanh_body(y_ref, g_ref, b_ref, o_ref):
    y = y_ref[...]
    n = y.shape[0]
    mean = jnp.sum(y, axis=0, keepdims=True) / n
    d = y - mean
    var = jnp.sum(d * d, axis=0, keepdims=True) / n
    xhat = d * jax.lax.rsqrt(var + _BN_EPS)
    o_ref[...] = jnp.tanh(xhat * g_ref[...] + b_ref[...])


def kernel(input, adj, W, bn_weight, bn_bias):
    n, din = input.shape
    dout = W.shape[1]

    support = pl.pallas_call(
        _support_body,
        out_shape=jax.ShapeDtypeStruct((n, dout), jnp.float32),
    )(input, W)

    nb = n // _BR
    out = pl.pallas_call(
        _spmm_body,
        grid=(nb,),
        in_specs=[
            pl.BlockSpec((_BR, n), lambda i: (i, 0)),
            pl.BlockSpec((n, dout), lambda i: (0, 0)),
        ],
        out_specs=pl.BlockSpec((_BR, dout), lambda i: (i, 0)),
        out_shape=jax.ShapeDtypeStruct((n, dout), jnp.float32),
    )(adj, support)

    g = bn_weight.reshape(1, dout)
    b = bn_bias.reshape(1, dout)
    return pl.pallas_call(
        _bn_tanh_body,
        out_shape=jax.ShapeDtypeStruct((n, dout), jnp.float32),
    )(out, g, b)


# speedup vs baseline: 1.1009x; 1.1009x over previous
import jax
import jax.numpy as jnp
from jax.experimental import pallas as pl
from jax.experimental.pallas import tpu as pltpu

_BN_EPS = 1e-5
_BR = 400


def _fused_body(x_ref, w_ref, adj_ref, g_ref, b_ref, o_ref,
                sup_ref, acc_ref, s1_ref, s2_ref):
    i = pl.program_id(0)
    nb = pl.num_programs(0) - 1
    n = acc_ref.shape[0]

    @pl.when(i == 0)
    def _():
        sup_ref[...] = jnp.dot(x_ref[...], w_ref[...],
                               preferred_element_type=jnp.float32)

    @pl.when(i < nb)
    def _():
        blk = jnp.dot(adj_ref[...], sup_ref[...],
                      preferred_element_type=jnp.float32)
        base = pl.multiple_of(i * _BR, _BR)
        acc_ref[pl.ds(base, _BR), :] = blk
        csum = jnp.sum(blk, axis=0, keepdims=True)
        csq = jnp.sum(blk * blk, axis=0, keepdims=True)

        @pl.when(i == 0)
        def _():
            s1_ref[...] = csum
            s2_ref[...] = csq

        @pl.when(i > 0)
        def _():
            s1_ref[...] += csum
            s2_ref[...] += csq

    @pl.when(i == nb)
    def _():
        mean = s1_ref[...] / n
        var = s2_ref[...] / n - mean * mean
        scale = g_ref[...] * jax.lax.rsqrt(var + _BN_EPS)
        shift = b_ref[...] - mean * scale
        o_ref[...] = jnp.tanh(acc_ref[...] * scale + shift)


def kernel(input, adj, W, bn_weight, bn_bias):
    n, din = input.shape
    dout = W.shape[1]
    nb = n // _BR
    g = bn_weight.reshape(1, dout)
    b = bn_bias.reshape(1, dout)

    return pl.pallas_call(
        _fused_body,
        grid=(nb + 1,),
        in_specs=[
            pl.BlockSpec((n, din), lambda i: (0, 0)),
            pl.BlockSpec((din, dout), lambda i: (0, 0)),
            pl.BlockSpec((_BR, n), lambda i: (jnp.minimum(i, nb - 1), 0)),
            pl.BlockSpec((1, dout), lambda i: (0, 0)),
            pl.BlockSpec((1, dout), lambda i: (0, 0)),
        ],
        out_specs=pl.BlockSpec((n, dout), lambda i: (0, 0)),
        out_shape=jax.ShapeDtypeStruct((n, dout), jnp.float32),
        scratch_shapes=[
            pltpu.VMEM((n, dout), jnp.float32),
            pltpu.VMEM((n, dout), jnp.float32),
            pltpu.VMEM((1, dout), jnp.float32),
            pltpu.VMEM((1, dout), jnp.float32),
        ],
    )(input, W, adj, g, b)
